# baseline (speedup 1.0000x reference)
import functools

import jax
import jax.numpy as jnp
from jax import lax
from jax.experimental import pallas as pl
from jax.experimental.pallas import tpu as pltpu
from jax.experimental.pallas import tpu_sc as plsc

_N = 10000
_E = 320000
_D = 128
_B = 1024

_NC = 2
_NS = 16
_NW = _NC * _NS
_EPW = _E // _NW
_K = 128
_EPWP = 10240
_NCH = _EPWP // _K
_NB = 5
_BCH = _NCH // _NB
_NPAD = _EPWP - _EPW
_NA = _N + _NPAD
_RA = 624
_TAIL = _N - _NS * _RA

_mesh = plsc.VectorSubcoreMesh(core_axis_name="c", subcore_axis_name="s")


_RZ = _RA // 3


def _zero_acc(zrows, acc, s):
    for j in range(3):
        pltpu.sync_copy(zrows, acc.at[pl.ds(s * _RA + j * _RZ, _RZ)])

    @pl.when(s == 0)
    def _zero_tail():
        pltpu.sync_copy(zrows.at[pl.ds(0, _TAIL)],
                        acc.at[pl.ds(_NS * _RA, _TAIL)])


def _copy_acc_out(acc, out_p, c, s):
    pltpu.sync_copy(acc.at[pl.ds(s * _RA, _RA)],
                    out_p.at[c, pl.ds(s * _RA, _RA)])

    @pl.when(s == 0)
    def _copy_tail():
        pltpu.sync_copy(acc.at[pl.ds(_NS * _RA, _TAIL)],
                        out_p.at[c, pl.ds(_NS * _RA, _TAIL)])


def _sc_agg_body(src4, dst4, h, zrows, out_p,
                 sI0, dI0, sI1, dI1, rows0, rows1,
                 acc, semi0, semi1, sem0, sem1):
    c = lax.axis_index("c")
    s = lax.axis_index("s")
    wid = c * _NS + s

    _zero_acc(zrows, acc, s)
    sbufs = (sI0, sI1)
    dbufs = (dI0, dI1)
    isems = (semi0, semi1)

    def _stage_start(b):
        pltpu.async_copy(src4.at[wid, b], sbufs[b % 2], isems[b % 2])
        pltpu.async_copy(dst4.at[wid, b], dbufs[b % 2], isems[b % 2])

    def _stage_wait(b):
        pltpu.make_async_copy(src4.at[wid, 0], sbufs[b % 2],
                              isems[b % 2]).wait()
        pltpu.make_async_copy(dst4.at[wid, 0], dbufs[b % 2],
                              isems[b % 2]).wait()

    _stage_start(0)
    plsc.subcore_barrier()

    def _gather_start(sIb, t, buf, sem):
        pltpu.async_copy(h.at[sIb.at[t]], buf, sem)

    def _gather_wait(sIb, buf, sem):
        pltpu.make_async_copy(h.at[sIb.at[0]], buf, sem).wait()

    def _scatter(dIb, t, buf):
        pltpu.sync_copy(buf, acc.at[dIb.at[t]], add=True)

    for b in range(_NB):
        sIb, dIb = sbufs[b % 2], dbufs[b % 2]
        if b + 1 < _NB:
            _stage_start(b + 1)
        _stage_wait(b)

        _gather_start(sIb, 0, rows0, sem0)

        def _pair(g, carry, sIb=sIb, dIb=dIb):
            t0 = g * 2
            _gather_start(sIb, t0 + 1, rows1, sem1)
            _gather_wait(sIb, rows0, sem0)
            _scatter(dIb, t0, rows0)
            _gather_start(sIb, t0 + 2, rows0, sem0)
            _gather_wait(sIb, rows1, sem1)
            _scatter(dIb, t0 + 1, rows1)
            return carry

        lax.fori_loop(0, (_BCH - 2) // 2, _pair, 0)
        _gather_start(sIb, _BCH - 1, rows1, sem1)
        _gather_wait(sIb, rows0, sem0)
        _scatter(dIb, _BCH - 2, rows0)
        _gather_wait(sIb, rows1, sem1)
        _scatter(dIb, _BCH - 1, rows1)

    plsc.subcore_barrier()
    _copy_acc_out(acc, out_p, c, s)


_sc_agg = pl.kernel(
    _sc_agg_body,
    out_type=(jax.ShapeDtypeStruct((_NC, _N, _D), jnp.float32),),
    mesh=_mesh,
    scratch_types=[
        pltpu.VMEM((_BCH, _K), jnp.int32),
        pltpu.VMEM((_BCH, _K), jnp.int32),
        pltpu.VMEM((_BCH, _K), jnp.int32),
        pltpu.VMEM((_BCH, _K), jnp.int32),
        pltpu.VMEM((_K, _D), jnp.float32),
        pltpu.VMEM((_K, _D), jnp.float32),
        pltpu.VMEM_SHARED((_NA, _D), jnp.float32),
        pltpu.SemaphoreType.DMA,
        pltpu.SemaphoreType.DMA,
        pltpu.SemaphoreType.DMA,
        pltpu.SemaphoreType.DMA,
    ],
)


def _sc_deg_body(dst4, ones_st, zrows, out_deg,
                 dI0, dI1, ones_v, acc, semi0, semi1):
    c = lax.axis_index("c")
    s = lax.axis_index("s")
    wid = c * _NS + s

    dbufs = (dI0, dI1)
    isems = (semi0, semi1)

    pltpu.async_copy(dst4.at[wid, 0], dI0, semi0)
    pltpu.sync_copy(ones_st, ones_v)
    _zero_acc(zrows, acc, s)
    plsc.subcore_barrier()

    for b in range(_NB):
        dIb = dbufs[b % 2]
        if b + 1 < _NB:
            pltpu.async_copy(dst4.at[wid, b + 1], dbufs[(b + 1) % 2],
                             isems[(b + 1) % 2])
        pltpu.make_async_copy(dst4.at[wid, 0], dIb, isems[b % 2]).wait()

        def _chunk(t, carry, dIb=dIb):
            pltpu.sync_copy(ones_v, acc.at[dIb.at[t]], add=True)
            return carry

        lax.fori_loop(0, _BCH, _chunk, 0)

    plsc.subcore_barrier()
    _copy_acc_out(acc, out_deg, c, s)


_sc_deg = pl.kernel(
    _sc_deg_body,
    out_type=(jax.ShapeDtypeStruct((_NC, _N, _D), jnp.float32),),
    mesh=_mesh,
    scratch_types=[
        pltpu.VMEM((_BCH, _K), jnp.int32),
        pltpu.VMEM((_BCH, _K), jnp.int32),
        pltpu.VMEM((_K, _D), jnp.float32),
        pltpu.VMEM_SHARED((_NA, _D), jnp.float32),
        pltpu.SemaphoreType.DMA,
        pltpu.SemaphoreType.DMA,
    ],
)


def _sc_gather_body(z, labels2d, batch, zb_out, lb_out,
                    bidx, zrows, lrows, sem):
    c = lax.axis_index("c")
    s = lax.axis_index("s")
    wid = c * _NS + s
    bpw = _B // _NW

    pltpu.sync_copy(batch.at[pl.ds(wid * bpw, bpw)], bidx)
    pltpu.async_copy(z.at[bidx], zrows, sem).wait()
    pltpu.sync_copy(zrows, zb_out.at[pl.ds(wid * bpw, bpw)])
    pltpu.async_copy(labels2d.at[bidx], lrows, sem).wait()
    pltpu.sync_copy(lrows, lb_out.at[pl.ds(wid * bpw, bpw)])


_sc_gather = pl.kernel(
    _sc_gather_body,
    out_type=(
        jax.ShapeDtypeStruct((_B, _D), jnp.float32),
        jax.ShapeDtypeStruct((_B, _D), jnp.int32),
    ),
    mesh=_mesh,
    scratch_types=[
        pltpu.VMEM((_B // _NW,), jnp.int32),
        pltpu.VMEM((_B // _NW, _D), jnp.float32),
        pltpu.VMEM((_B // _NW, _D), jnp.int32),
        pltpu.SemaphoreType.DMA,
    ],
)


_BN = 1000


def _tc_layer1_kernel(p0, p1, degp, h, wl, b, wr, out_h, out_rdeg):
    deg = jnp.sum(degp[...], axis=0)[:, 0:1]
    rdeg = 1.0 / jnp.maximum(deg, 1.0)
    out_rdeg[...] = rdeg
    agg = (p0[...] + p1[...]) * rdeg
    r = (jnp.dot(agg, wl[...], preferred_element_type=jnp.float32)
         + b[...]
         + jnp.dot(h[...], wr[...], preferred_element_type=jnp.float32))
    out_h[...] = jnp.maximum(r, 0.0)


def _tc_layerN_kernel(relu, p0, p1, rdeg_in, h, wl, b, wr, out_h):
    rdeg = rdeg_in[...]
    agg = (p0[...] + p1[...]) * rdeg
    r = (jnp.dot(agg, wl[...], preferred_element_type=jnp.float32)
         + b[...]
         + jnp.dot(h[...], wr[...], preferred_element_type=jnp.float32))
    out_h[...] = jnp.maximum(r, 0.0) if relu else r


_row_spec = pl.BlockSpec((_BN, _D), lambda i: (i, 0))
_w_spec = pl.BlockSpec((_D, _D), lambda i: (0, 0))
_b_spec = pl.BlockSpec((1, _D), lambda i: (0, 0))
_rdeg_spec = pl.BlockSpec((_BN, 1), lambda i: (i, 0))

_tc_layer1 = pl.pallas_call(
    _tc_layer1_kernel,
    grid=(_N // _BN,),
    in_specs=[
        _row_spec, _row_spec,
        pl.BlockSpec((_NC, _BN, _D), lambda i: (0, i, 0)),
        _row_spec, _w_spec, _b_spec, _w_spec,
    ],
    out_specs=[_row_spec, _rdeg_spec],
    out_shape=[
        jax.ShapeDtypeStruct((_N, _D), jnp.float32),
        jax.ShapeDtypeStruct((_N, 1), jnp.float32),
    ],
)


def _make_tc_layer(relu):
    return pl.pallas_call(
        functools.partial(_tc_layerN_kernel, relu),
        grid=(_N // _BN,),
        in_specs=[
            _row_spec, _row_spec, _rdeg_spec,
            _row_spec, _w_spec, _b_spec, _w_spec,
        ],
        out_specs=_row_spec,
        out_shape=jax.ShapeDtypeStruct((_N, _D), jnp.float32),
    )


_tc_layer2 = _make_tc_layer(True)
_tc_layer3 = _make_tc_layer(False)


def _tc_loss_kernel(zb, lb, wlin, blin, out):
    logits = (jnp.dot(zb[...], wlin[...], preferred_element_type=jnp.float32)
              + blin[...])
    col = lax.broadcasted_iota(jnp.int32, (_B, _D), 1)
    valid = col < 2
    neg = jnp.float32(-1e30)
    lmax = jnp.max(jnp.where(valid, logits, neg), axis=1, keepdims=True)
    ex = jnp.where(valid, jnp.exp(logits - lmax), 0.0)
    p = ex / jnp.sum(ex, axis=1, keepdims=True)
    pmax = jnp.max(jnp.where(valid, p, neg), axis=1, keepdims=True)
    ex2 = jnp.where(valid, jnp.exp(p - pmax), 0.0)
    lse2 = jnp.log(jnp.sum(ex2, axis=1, keepdims=True)) + pmax
    logp = p - lse2
    sel = jnp.sum(jnp.where(col == lb[...][:, 0:1], logp, 0.0), axis=1)
    loss = -jnp.sum(sel) / jnp.float32(_B)
    out[...] = jnp.full((8, 128), loss, jnp.float32)


_tc_loss = pl.pallas_call(
    _tc_loss_kernel,
    out_shape=jax.ShapeDtypeStruct((8, 128), jnp.float32),
)


def kernel(x, ei, batch, labels, W1l, b1, W1r, W2l, b2, W2r, W3l, b3, W3r,
           Wlin, blin):
    ei = ei.astype(jnp.int32)
    batch = batch.astype(jnp.int32)
    labels = labels.astype(jnp.int32)

    zrows_stage = jnp.zeros((_RZ, _D), jnp.float32)
    ones_stage = jnp.ones((_K, _D), jnp.float32)

    w1l = W1l.T
    w1r = W1r.T
    w2l = W2l.T
    w2r = W2r.T
    w3l = W3l.T
    w3r = W3r.T
    wlin = jnp.zeros((_D, _D), jnp.float32).at[:, :2].set(Wlin.T)
    blin_p = jnp.zeros((1, _D), jnp.float32).at[0, :2].set(blin)
    b1r = b1.reshape(1, _D)
    b2r = b2.reshape(1, _D)
    b3r = b3.reshape(1, _D)

    trash = _N + jnp.arange(_NPAD, dtype=jnp.int32)
    src = jnp.concatenate(
        [ei[0].reshape(_NW, _EPW),
         jnp.zeros((_NW, _NPAD), jnp.int32)],
        axis=1).reshape(_NW, _NB, _BCH, _K)
    dst = jnp.concatenate(
        [ei[1].reshape(_NW, _EPW),
         jnp.broadcast_to(trash, (_NW, _NPAD))],
        axis=1).reshape(_NW, _NB, _BCH, _K)
    (degp,) = _sc_deg(dst, ones_stage, zrows_stage)
    (p1,) = _sc_agg(src, dst, x, zrows_stage)
    h1, rdeg = _tc_layer1(p1[0], p1[1], degp, x, w1l, b1r, w1r)

    (p2,) = _sc_agg(src, dst, h1, zrows_stage)
    h2 = _tc_layer2(p2[0], p2[1], rdeg, h1, w2l, b2r, w2r)

    (p3,) = _sc_agg(src, dst, h2, zrows_stage)
    z = _tc_layer3(p3[0], p3[1], rdeg, h2, w3l, b3r, w3r)

    labels2d = jnp.broadcast_to(labels[:, None], (_N, _D))
    zb, lb = _sc_gather(z, labels2d, batch)
    loss = _tc_loss(zb, lb, wlin, blin_p)
    return loss[0, 0]


# speedup vs baseline: 2.4489x; 2.4489x over previous
import functools

import jax
import jax.numpy as jnp
from jax import lax
from jax.experimental import pallas as pl
from jax.experimental.pallas import tpu as pltpu
from jax.experimental.pallas import tpu_sc as plsc

_N = 10000
_E = 320000
_D = 128
_B = 1024

_NC = 2
_NS = 16
_NW = _NC * _NS
_EPW = _E // _NW
_K = 80
_EPWP = 10000
_NCH = _EPWP // _K
_NB = 5
_BCH = _NCH // _NB
_NPAD = _EPWP - _EPW
_NA = _N + _NPAD
_RA = 624
_TAIL = _N - _NS * _RA

_mesh = plsc.VectorSubcoreMesh(core_axis_name="c", subcore_axis_name="s")


_RZ = _RA // 3


def _zero_acc(zrows, acc, s):
    for j in range(3):
        pltpu.sync_copy(zrows, acc.at[pl.ds(s * _RA + j * _RZ, _RZ)])

    @pl.when(s == 0)
    def _zero_tail():
        pltpu.sync_copy(zrows.at[pl.ds(0, _TAIL)],
                        acc.at[pl.ds(_NS * _RA, _TAIL)])


def _copy_acc_out(acc, out_p, c, s):
    pltpu.sync_copy(acc.at[pl.ds(s * _RA, _RA)],
                    out_p.at[c, pl.ds(s * _RA, _RA)])

    @pl.when(s == 0)
    def _copy_tail():
        pltpu.sync_copy(acc.at[pl.ds(_NS * _RA, _TAIL)],
                        out_p.at[c, pl.ds(_NS * _RA, _TAIL)])


def _sc_agg_body(src4, dst4, h, zrows, out_p,
                 sI0, dI0, sI1, dI1, rows0, rows1,
                 acc, semi0, semi1, sem0, sem1):
    c = lax.axis_index("c")
    s = lax.axis_index("s")
    wid = c * _NS + s

    _zero_acc(zrows, acc, s)
    sbufs = (sI0, sI1)
    dbufs = (dI0, dI1)
    isems = (semi0, semi1)

    def _stage_start(b):
        pltpu.async_copy(src4.at[wid, b], sbufs[b % 2], isems[b % 2])
        pltpu.async_copy(dst4.at[wid, b], dbufs[b % 2], isems[b % 2])

    def _stage_wait(b):
        pltpu.make_async_copy(src4.at[wid, 0], sbufs[b % 2],
                              isems[b % 2]).wait()
        pltpu.make_async_copy(dst4.at[wid, 0], dbufs[b % 2],
                              isems[b % 2]).wait()

    _stage_start(0)
    plsc.subcore_barrier()

    def _gather_start(sIb, t, buf, sem):
        pltpu.async_copy(h.at[sIb.at[t]], buf, sem)

    def _gather_wait(sIb, buf, sem):
        pltpu.make_async_copy(h.at[sIb.at[0]], buf, sem).wait()

    def _scatter(dIb, t, buf):
        pltpu.sync_copy(buf, acc.at[dIb.at[t]], add=True)

    for b in range(_NB):
        sIb, dIb = sbufs[b % 2], dbufs[b % 2]
        if b + 1 < _NB:
            _stage_start(b + 1)
        _stage_wait(b)

        _gather_start(sIb, 0, rows0, sem0)

        def _pair(g, carry, sIb=sIb, dIb=dIb):
            t0 = g * 2
            _gather_start(sIb, t0 + 1, rows1, sem1)
            _gather_wait(sIb, rows0, sem0)
            _scatter(dIb, t0, rows0)
            _gather_start(sIb, t0 + 2, rows0, sem0)
            _gather_wait(sIb, rows1, sem1)
            _scatter(dIb, t0 + 1, rows1)
            return carry

        if _BCH % 2:
            lax.fori_loop(0, (_BCH - 1) // 2, _pair, 0)
            _gather_wait(sIb, rows0, sem0)
            _scatter(dIb, _BCH - 1, rows0)
        else:
            lax.fori_loop(0, (_BCH - 2) // 2, _pair, 0)
            _gather_start(sIb, _BCH - 1, rows1, sem1)
            _gather_wait(sIb, rows0, sem0)
            _scatter(dIb, _BCH - 2, rows0)
            _gather_wait(sIb, rows1, sem1)
            _scatter(dIb, _BCH - 1, rows1)

    plsc.subcore_barrier()
    _copy_acc_out(acc, out_p, c, s)


_sc_agg = pl.kernel(
    _sc_agg_body,
    out_type=(jax.ShapeDtypeStruct((_NC, _N, _D), jnp.float32),),
    mesh=_mesh,
    scratch_types=[
        pltpu.VMEM((_BCH, _K), jnp.int32),
        pltpu.VMEM((_BCH, _K), jnp.int32),
        pltpu.VMEM((_BCH, _K), jnp.int32),
        pltpu.VMEM((_BCH, _K), jnp.int32),
        pltpu.VMEM((_K, _D), jnp.float32),
        pltpu.VMEM((_K, _D), jnp.float32),
        pltpu.VMEM_SHARED((_NA, _D), jnp.float32),
        pltpu.SemaphoreType.DMA,
        pltpu.SemaphoreType.DMA,
        pltpu.SemaphoreType.DMA,
        pltpu.SemaphoreType.DMA,
    ],
)


def _sc_deg_body(dst4, ones_st, zrows, out_deg,
                 dI0, dI1, ones_v, acc, semi0, semi1):
    c = lax.axis_index("c")
    s = lax.axis_index("s")
    wid = c * _NS + s

    dbufs = (dI0, dI1)
    isems = (semi0, semi1)

    pltpu.async_copy(dst4.at[wid, 0], dI0, semi0)
    pltpu.sync_copy(ones_st, ones_v)
    _zero_acc(zrows, acc, s)
    plsc.subcore_barrier()

    for b in range(_NB):
        dIb = dbufs[b % 2]
        if b + 1 < _NB:
            pltpu.async_copy(dst4.at[wid, b + 1], dbufs[(b + 1) % 2],
                             isems[(b + 1) % 2])
        pltpu.make_async_copy(dst4.at[wid, 0], dIb, isems[b % 2]).wait()

        def _chunk(t, carry, dIb=dIb):
            pltpu.sync_copy(ones_v, acc.at[dIb.at[t]], add=True)
            return carry

        lax.fori_loop(0, _BCH, _chunk, 0)

    plsc.subcore_barrier()
    _copy_acc_out(acc, out_deg, c, s)


_sc_deg = pl.kernel(
    _sc_deg_body,
    out_type=(jax.ShapeDtypeStruct((_NC, _N, _D), jnp.float32),),
    mesh=_mesh,
    scratch_types=[
        pltpu.VMEM((_BCH, _K), jnp.int32),
        pltpu.VMEM((_BCH, _K), jnp.int32),
        pltpu.VMEM((_K, _D), jnp.float32),
        pltpu.VMEM_SHARED((_NA, _D), jnp.float32),
        pltpu.SemaphoreType.DMA,
        pltpu.SemaphoreType.DMA,
    ],
)


def _sc_gather_body(z, labels2d, batch, zb_out, lb_out,
                    bidx, zrows, lrows, sem):
    c = lax.axis_index("c")
    s = lax.axis_index("s")
    wid = c * _NS + s
    bpw = _B // _NW

    pltpu.sync_copy(batch.at[pl.ds(wid * bpw, bpw)], bidx)
    pltpu.async_copy(z.at[bidx], zrows, sem).wait()
    pltpu.sync_copy(zrows, zb_out.at[pl.ds(wid * bpw, bpw)])
    pltpu.async_copy(labels2d.at[bidx], lrows, sem).wait()
    pltpu.sync_copy(lrows, lb_out.at[pl.ds(wid * bpw, bpw)])


_sc_gather = pl.kernel(
    _sc_gather_body,
    out_type=(
        jax.ShapeDtypeStruct((_B, _D), jnp.float32),
        jax.ShapeDtypeStruct((_B, _D), jnp.int32),
    ),
    mesh=_mesh,
    scratch_types=[
        pltpu.VMEM((_B // _NW,), jnp.int32),
        pltpu.VMEM((_B // _NW, _D), jnp.float32),
        pltpu.VMEM((_B // _NW, _D), jnp.int32),
        pltpu.SemaphoreType.DMA,
    ],
)


_BN = 1000


def _tc_layer1_kernel(p0, p1, degp, h, wl, b, wr, out_h, out_rdeg):
    deg = jnp.sum(degp[...], axis=0)[:, 0:1]
    rdeg = 1.0 / jnp.maximum(deg, 1.0)
    out_rdeg[...] = rdeg
    agg = (p0[...] + p1[...]) * rdeg
    r = (jnp.dot(agg, wl[...], preferred_element_type=jnp.float32)
         + b[...]
         + jnp.dot(h[...], wr[...], preferred_element_type=jnp.float32))
    out_h[...] = jnp.maximum(r, 0.0)


def _tc_layerN_kernel(relu, p0, p1, rdeg_in, h, wl, b, wr, out_h):
    rdeg = rdeg_in[...]
    agg = (p0[...] + p1[...]) * rdeg
    r = (jnp.dot(agg, wl[...], preferred_element_type=jnp.float32)
         + b[...]
         + jnp.dot(h[...], wr[...], preferred_element_type=jnp.float32))
    out_h[...] = jnp.maximum(r, 0.0) if relu else r


_row_spec = pl.BlockSpec((_BN, _D), lambda i: (i, 0))
_w_spec = pl.BlockSpec((_D, _D), lambda i: (0, 0))
_b_spec = pl.BlockSpec((1, _D), lambda i: (0, 0))
_rdeg_spec = pl.BlockSpec((_BN, 1), lambda i: (i, 0))

_tc_layer1 = pl.pallas_call(
    _tc_layer1_kernel,
    grid=(_N // _BN,),
    in_specs=[
        _row_spec, _row_spec,
        pl.BlockSpec((_NC, _BN, _D), lambda i: (0, i, 0)),
        _row_spec, _w_spec, _b_spec, _w_spec,
    ],
    out_specs=[_row_spec, _rdeg_spec],
    out_shape=[
        jax.ShapeDtypeStruct((_N, _D), jnp.float32),
        jax.ShapeDtypeStruct((_N, 1), jnp.float32),
    ],
)


def _make_tc_layer(relu):
    return pl.pallas_call(
        functools.partial(_tc_layerN_kernel, relu),
        grid=(_N // _BN,),
        in_specs=[
            _row_spec, _row_spec, _rdeg_spec,
            _row_spec, _w_spec, _b_spec, _w_spec,
        ],
        out_specs=_row_spec,
        out_shape=jax.ShapeDtypeStruct((_N, _D), jnp.float32),
    )


_tc_layer2 = _make_tc_layer(True)
_tc_layer3 = _make_tc_layer(False)


def _tc_loss_kernel(zb, lb, wlin, blin, out):
    logits = (jnp.dot(zb[...], wlin[...], preferred_element_type=jnp.float32)
              + blin[...])
    col = lax.broadcasted_iota(jnp.int32, (_B, _D), 1)
    valid = col < 2
    neg = jnp.float32(-1e30)
    lmax = jnp.max(jnp.where(valid, logits, neg), axis=1, keepdims=True)
    ex = jnp.where(valid, jnp.exp(logits - lmax), 0.0)
    p = ex / jnp.sum(ex, axis=1, keepdims=True)
    pmax = jnp.max(jnp.where(valid, p, neg), axis=1, keepdims=True)
    ex2 = jnp.where(valid, jnp.exp(p - pmax), 0.0)
    lse2 = jnp.log(jnp.sum(ex2, axis=1, keepdims=True)) + pmax
    logp = p - lse2
    sel = jnp.sum(jnp.where(col == lb[...][:, 0:1], logp, 0.0), axis=1)
    loss = -jnp.sum(sel) / jnp.float32(_B)
    out[...] = jnp.full((8, 128), loss, jnp.float32)


_tc_loss = pl.pallas_call(
    _tc_loss_kernel,
    out_shape=jax.ShapeDtypeStruct((8, 128), jnp.float32),
)


def kernel(x, ei, batch, labels, W1l, b1, W1r, W2l, b2, W2r, W3l, b3, W3r,
           Wlin, blin):
    ei = ei.astype(jnp.int32)
    batch = batch.astype(jnp.int32)
    labels = labels.astype(jnp.int32)

    zrows_stage = jnp.zeros((_RZ, _D), jnp.float32)
    ones_stage = jnp.ones((_K, _D), jnp.float32)

    w1l = W1l.T
    w1r = W1r.T
    w2l = W2l.T
    w2r = W2r.T
    w3l = W3l.T
    w3r = W3r.T
    wlin = jnp.zeros((_D, _D), jnp.float32).at[:, :2].set(Wlin.T)
    blin_p = jnp.zeros((1, _D), jnp.float32).at[0, :2].set(blin)
    b1r = b1.reshape(1, _D)
    b2r = b2.reshape(1, _D)
    b3r = b3.reshape(1, _D)

    trash = _N + jnp.arange(_NPAD, dtype=jnp.int32)
    src = jnp.concatenate(
        [ei[0].reshape(_NW, _EPW),
         jnp.zeros((_NW, _NPAD), jnp.int32)],
        axis=1).reshape(_NW, _NB, _BCH, _K)
    dst = jnp.concatenate(
        [ei[1].reshape(_NW, _EPW),
         jnp.broadcast_to(trash, (_NW, _NPAD))],
        axis=1).reshape(_NW, _NB, _BCH, _K)
    (degp,) = _sc_deg(dst, ones_stage, zrows_stage)
    (p1,) = _sc_agg(src, dst, x, zrows_stage)
    h1, rdeg = _tc_layer1(p1[0], p1[1], degp, x, w1l, b1r, w1r)

    (p2,) = _sc_agg(src, dst, h1, zrows_stage)
    h2 = _tc_layer2(p2[0], p2[1], rdeg, h1, w2l, b2r, w2r)

    (p3,) = _sc_agg(src, dst, h2, zrows_stage)
    z = _tc_layer3(p3[0], p3[1], rdeg, h2, w3l, b3r, w3r)

    labels2d = jnp.broadcast_to(labels[:, None], (_N, _D))
    zb, lb = _sc_gather(z, labels2d, batch)
    loss = _tc_loss(zb, lb, wlin, blin_p)
    return loss[0, 0]


# speedup vs baseline: 2.7188x; 1.1102x over previous
import functools

import jax
import jax.numpy as jnp
from jax import lax
from jax.experimental import pallas as pl
from jax.experimental.pallas import tpu as pltpu
from jax.experimental.pallas import tpu_sc as plsc

_N = 10000
_E = 320000
_D = 128
_B = 1024

_NC = 2
_NS = 16
_NW = _NC * _NS
_EPW = _E // _NW
_K = 80
_EPWP = 10000
_NCH = _EPWP // _K
_NB = 5
_BCH = _NCH // _NB
_NPAD = _EPWP - _EPW
_NA = _N + _NPAD
_RA = 624
_TAIL = _N - _NS * _RA

_mesh = plsc.VectorSubcoreMesh(core_axis_name="c", subcore_axis_name="s")


_RZ = _RA // 3


def _zero_acc(zrows, acc, s):
    for j in range(3):
        pltpu.sync_copy(zrows, acc.at[pl.ds(s * _RA + j * _RZ, _RZ)])

    @pl.when(s == 0)
    def _zero_tail():
        pltpu.sync_copy(zrows.at[pl.ds(0, _TAIL)],
                        acc.at[pl.ds(_NS * _RA, _TAIL)])


def _copy_acc_out(acc, out_p, c, s):
    pltpu.sync_copy(acc.at[pl.ds(s * _RA, _RA)],
                    out_p.at[c, pl.ds(s * _RA, _RA)])

    @pl.when(s == 0)
    def _copy_tail():
        pltpu.sync_copy(acc.at[pl.ds(_NS * _RA, _TAIL)],
                        out_p.at[c, pl.ds(_NS * _RA, _TAIL)])


def _sc_agg_body(src4, dst4, h, zrows, out_p,
                 sI0, dI0, sI1, dI1, rows0, rows1, rows2,
                 acc, semi0, semi1, sem0, sem1, sem2):
    c = lax.axis_index("c")
    s = lax.axis_index("s")
    wid = c * _NS + s

    _zero_acc(zrows, acc, s)
    sbufs = (sI0, sI1)
    dbufs = (dI0, dI1)
    isems = (semi0, semi1)

    def _stage_start(b):
        pltpu.async_copy(src4.at[wid, b], sbufs[b % 2], isems[b % 2])
        pltpu.async_copy(dst4.at[wid, b], dbufs[b % 2], isems[b % 2])

    def _stage_wait(b):
        pltpu.make_async_copy(src4.at[wid, 0], sbufs[b % 2],
                              isems[b % 2]).wait()
        pltpu.make_async_copy(dst4.at[wid, 0], dbufs[b % 2],
                              isems[b % 2]).wait()

    _stage_start(0)
    plsc.subcore_barrier()

    def _gather_start(sIb, t, buf, sem):
        pltpu.async_copy(h.at[sIb.at[t]], buf, sem)

    def _gather_wait(sIb, buf, sem):
        pltpu.make_async_copy(h.at[sIb.at[0]], buf, sem).wait()

    def _scatter(dIb, t, buf):
        pltpu.sync_copy(buf, acc.at[dIb.at[t]], add=True)

    rows = (rows0, rows1, rows2)
    sems = (sem0, sem1, sem2)

    for b in range(_NB):
        sIb, dIb = sbufs[b % 2], dbufs[b % 2]
        if b + 1 < _NB:
            _stage_start(b + 1)
        _stage_wait(b)

        _gather_start(sIb, 0, rows0, sem0)
        _gather_start(sIb, 1, rows1, sem1)

        def _triple(g, carry, sIb=sIb, dIb=dIb):
            t0 = g * 3
            for j in range(3):
                _gather_start(sIb, t0 + 2 + j, rows[(2 + j) % 3],
                              sems[(2 + j) % 3])
                _gather_wait(sIb, rows[j], sems[j])
                _scatter(dIb, t0 + j, rows[j])
            return carry

        lax.fori_loop(0, (_BCH - 4) // 3, _triple, 0)
        e = _BCH - 4
        _gather_start(sIb, e + 2, rows[(e + 2) % 3], sems[(e + 2) % 3])
        _gather_wait(sIb, rows[e % 3], sems[e % 3])
        _scatter(dIb, e, rows[e % 3])
        _gather_start(sIb, e + 3, rows[(e + 3) % 3], sems[(e + 3) % 3])
        for j in range(1, 4):
            _gather_wait(sIb, rows[(e + j) % 3], sems[(e + j) % 3])
            _scatter(dIb, e + j, rows[(e + j) % 3])

    plsc.subcore_barrier()
    _copy_acc_out(acc, out_p, c, s)


_sc_agg = pl.kernel(
    _sc_agg_body,
    out_type=(jax.ShapeDtypeStruct((_NC, _N, _D), jnp.float32),),
    mesh=_mesh,
    scratch_types=[
        pltpu.VMEM((_BCH, _K), jnp.int32),
        pltpu.VMEM((_BCH, _K), jnp.int32),
        pltpu.VMEM((_BCH, _K), jnp.int32),
        pltpu.VMEM((_BCH, _K), jnp.int32),
        pltpu.VMEM((_K, _D), jnp.float32),
        pltpu.VMEM((_K, _D), jnp.float32),
        pltpu.VMEM((_K, _D), jnp.float32),
        pltpu.VMEM_SHARED((_NA, _D), jnp.float32),
        pltpu.SemaphoreType.DMA,
        pltpu.SemaphoreType.DMA,
        pltpu.SemaphoreType.DMA,
        pltpu.SemaphoreType.DMA,
        pltpu.SemaphoreType.DMA,
    ],
)


def _sc_deg_body(dst4, ones_st, zrows, out_deg,
                 dI0, dI1, ones_v, acc, semi0, semi1):
    c = lax.axis_index("c")
    s = lax.axis_index("s")
    wid = c * _NS + s

    dbufs = (dI0, dI1)
    isems = (semi0, semi1)

    pltpu.async_copy(dst4.at[wid, 0], dI0, semi0)
    pltpu.sync_copy(ones_st, ones_v)
    _zero_acc(zrows, acc, s)
    plsc.subcore_barrier()

    for b in range(_NB):
        dIb = dbufs[b % 2]
        if b + 1 < _NB:
            pltpu.async_copy(dst4.at[wid, b + 1], dbufs[(b + 1) % 2],
                             isems[(b + 1) % 2])
        pltpu.make_async_copy(dst4.at[wid, 0], dIb, isems[b % 2]).wait()

        def _chunk(t, carry, dIb=dIb):
            pltpu.sync_copy(ones_v, acc.at[dIb.at[t]], add=True)
            return carry

        lax.fori_loop(0, _BCH, _chunk, 0)

    plsc.subcore_barrier()
    _copy_acc_out(acc, out_deg, c, s)


_sc_deg = pl.kernel(
    _sc_deg_body,
    out_type=(jax.ShapeDtypeStruct((_NC, _N, _D), jnp.float32),),
    mesh=_mesh,
    scratch_types=[
        pltpu.VMEM((_BCH, _K), jnp.int32),
        pltpu.VMEM((_BCH, _K), jnp.int32),
        pltpu.VMEM((_K, _D), jnp.float32),
        pltpu.VMEM_SHARED((_NA, _D), jnp.float32),
        pltpu.SemaphoreType.DMA,
        pltpu.SemaphoreType.DMA,
    ],
)


def _sc_gather_body(z, labels2d, batch, zb_out, lb_out,
                    bidx, zrows, lrows, sem):
    c = lax.axis_index("c")
    s = lax.axis_index("s")
    wid = c * _NS + s
    bpw = _B // _NW

    pltpu.sync_copy(batch.at[pl.ds(wid * bpw, bpw)], bidx)
    pltpu.async_copy(z.at[bidx], zrows, sem).wait()
    pltpu.sync_copy(zrows, zb_out.at[pl.ds(wid * bpw, bpw)])
    pltpu.async_copy(labels2d.at[bidx], lrows, sem).wait()
    pltpu.sync_copy(lrows, lb_out.at[pl.ds(wid * bpw, bpw)])


_sc_gather = pl.kernel(
    _sc_gather_body,
    out_type=(
        jax.ShapeDtypeStruct((_B, _D), jnp.float32),
        jax.ShapeDtypeStruct((_B, _D), jnp.int32),
    ),
    mesh=_mesh,
    scratch_types=[
        pltpu.VMEM((_B // _NW,), jnp.int32),
        pltpu.VMEM((_B // _NW, _D), jnp.float32),
        pltpu.VMEM((_B // _NW, _D), jnp.int32),
        pltpu.SemaphoreType.DMA,
    ],
)


_BN = 1000


def _tc_layer1_kernel(p0, p1, degp, h, wl, b, wr, out_h, out_rdeg):
    deg = jnp.sum(degp[...], axis=0)[:, 0:1]
    rdeg = 1.0 / jnp.maximum(deg, 1.0)
    out_rdeg[...] = rdeg
    agg = (p0[...] + p1[...]) * rdeg
    r = (jnp.dot(agg, wl[...], preferred_element_type=jnp.float32)
         + b[...]
         + jnp.dot(h[...], wr[...], preferred_element_type=jnp.float32))
    out_h[...] = jnp.maximum(r, 0.0)


def _tc_layerN_kernel(relu, p0, p1, rdeg_in, h, wl, b, wr, out_h):
    rdeg = rdeg_in[...]
    agg = (p0[...] + p1[...]) * rdeg
    r = (jnp.dot(agg, wl[...], preferred_element_type=jnp.float32)
         + b[...]
         + jnp.dot(h[...], wr[...], preferred_element_type=jnp.float32))
    out_h[...] = jnp.maximum(r, 0.0) if relu else r


_row_spec = pl.BlockSpec((_BN, _D), lambda i: (i, 0))
_w_spec = pl.BlockSpec((_D, _D), lambda i: (0, 0))
_b_spec = pl.BlockSpec((1, _D), lambda i: (0, 0))
_rdeg_spec = pl.BlockSpec((_BN, 1), lambda i: (i, 0))

_tc_layer1 = pl.pallas_call(
    _tc_layer1_kernel,
    grid=(_N // _BN,),
    in_specs=[
        _row_spec, _row_spec,
        pl.BlockSpec((_NC, _BN, _D), lambda i: (0, i, 0)),
        _row_spec, _w_spec, _b_spec, _w_spec,
    ],
    out_specs=[_row_spec, _rdeg_spec],
    out_shape=[
        jax.ShapeDtypeStruct((_N, _D), jnp.float32),
        jax.ShapeDtypeStruct((_N, 1), jnp.float32),
    ],
)


def _make_tc_layer(relu):
    return pl.pallas_call(
        functools.partial(_tc_layerN_kernel, relu),
        grid=(_N // _BN,),
        in_specs=[
            _row_spec, _row_spec, _rdeg_spec,
            _row_spec, _w_spec, _b_spec, _w_spec,
        ],
        out_specs=_row_spec,
        out_shape=jax.ShapeDtypeStruct((_N, _D), jnp.float32),
    )


_tc_layer2 = _make_tc_layer(True)
_tc_layer3 = _make_tc_layer(False)


def _tc_loss_kernel(zb, lb, wlin, blin, out):
    logits = (jnp.dot(zb[...], wlin[...], preferred_element_type=jnp.float32)
              + blin[...])
    col = lax.broadcasted_iota(jnp.int32, (_B, _D), 1)
    valid = col < 2
    neg = jnp.float32(-1e30)
    lmax = jnp.max(jnp.where(valid, logits, neg), axis=1, keepdims=True)
    ex = jnp.where(valid, jnp.exp(logits - lmax), 0.0)
    p = ex / jnp.sum(ex, axis=1, keepdims=True)
    pmax = jnp.max(jnp.where(valid, p, neg), axis=1, keepdims=True)
    ex2 = jnp.where(valid, jnp.exp(p - pmax), 0.0)
    lse2 = jnp.log(jnp.sum(ex2, axis=1, keepdims=True)) + pmax
    logp = p - lse2
    sel = jnp.sum(jnp.where(col == lb[...][:, 0:1], logp, 0.0), axis=1)
    loss = -jnp.sum(sel) / jnp.float32(_B)
    out[...] = jnp.full((8, 128), loss, jnp.float32)


_tc_loss = pl.pallas_call(
    _tc_loss_kernel,
    out_shape=jax.ShapeDtypeStruct((8, 128), jnp.float32),
)


def kernel(x, ei, batch, labels, W1l, b1, W1r, W2l, b2, W2r, W3l, b3, W3r,
           Wlin, blin):
    ei = ei.astype(jnp.int32)
    batch = batch.astype(jnp.int32)
    labels = labels.astype(jnp.int32)

    zrows_stage = jnp.zeros((_RZ, _D), jnp.float32)
    ones_stage = jnp.ones((_K, _D), jnp.float32)

    w1l = W1l.T
    w1r = W1r.T
    w2l = W2l.T
    w2r = W2r.T
    w3l = W3l.T
    w3r = W3r.T
    wlin = jnp.zeros((_D, _D), jnp.float32).at[:, :2].set(Wlin.T)
    blin_p = jnp.zeros((1, _D), jnp.float32).at[0, :2].set(blin)
    b1r = b1.reshape(1, _D)
    b2r = b2.reshape(1, _D)
    b3r = b3.reshape(1, _D)

    trash = _N + jnp.arange(_NPAD, dtype=jnp.int32)
    src = jnp.concatenate(
        [ei[0].reshape(_NW, _EPW),
         jnp.zeros((_NW, _NPAD), jnp.int32)],
        axis=1).reshape(_NW, _NB, _BCH, _K)
    dst = jnp.concatenate(
        [ei[1].reshape(_NW, _EPW),
         jnp.broadcast_to(trash, (_NW, _NPAD))],
        axis=1).reshape(_NW, _NB, _BCH, _K)
    (degp,) = _sc_deg(dst, ones_stage, zrows_stage)
    (p1,) = _sc_agg(src, dst, x, zrows_stage)
    h1, rdeg = _tc_layer1(p1[0], p1[1], degp, x, w1l, b1r, w1r)

    (p2,) = _sc_agg(src, dst, h1, zrows_stage)
    h2 = _tc_layer2(p2[0], p2[1], rdeg, h1, w2l, b2r, w2r)

    (p3,) = _sc_agg(src, dst, h2, zrows_stage)
    z = _tc_layer3(p3[0], p3[1], rdeg, h2, w3l, b3r, w3r)

    labels2d = jnp.broadcast_to(labels[:, None], (_N, _D))
    zb, lb = _sc_gather(z, labels2d, batch)
    loss = _tc_loss(zb, lb, wlin, blin_p)
    return loss[0, 0]
